# Initial kernel scaffold; baseline (speedup 1.0000x reference)
import jax
import jax.numpy as jnp
from jax.experimental import pallas as pl


def kernel(x, A, phenotypes, W, b, temperature):
    raise NotImplementedError("write your pallas kernel here")


# speedup vs baseline: 8.4349x; 8.4349x over previous
import functools

import jax
import jax.numpy as jnp
from jax.experimental import pallas as pl
from jax.experimental.pallas import tpu as pltpu

_K = 16


def _threefry_bits(flat):
    u32 = jnp.uint32
    k1 = u32(0)
    k2 = u32(42)
    ks2 = u32(0x1BD11BDA) ^ k1 ^ k2
    ks = [k1, k2, ks2]
    r0 = (13, 15, 26, 6)
    r1 = (17, 29, 16, 24)

    x0 = jnp.full_like(flat, k1)
    x1 = flat + k2

    def rounds(x0, x1, rots):
        for r in rots:
            x0 = x0 + x1
            x1 = ((x1 << u32(r)) | (x1 >> u32(32 - r))) ^ x0
        return x0, x1

    x0, x1 = rounds(x0, x1, r0)
    x0 = x0 + ks[1]
    x1 = x1 + ks[2] + u32(1)
    x0, x1 = rounds(x0, x1, r1)
    x0 = x0 + ks[2]
    x1 = x1 + ks[0] + u32(2)
    x0, x1 = rounds(x0, x1, r0)
    x0 = x0 + ks[0]
    x1 = x1 + ks[1] + u32(3)
    x0, x1 = rounds(x0, x1, r1)
    x0 = x0 + ks[1]
    x1 = x1 + ks[2] + u32(4)
    x0, x1 = rounds(x0, x1, r0)
    x0 = x0 + ks[2]
    x1 = x1 + ks[0] + u32(5)
    return x0 ^ x1


def _embed_kernel(phen_ref, w_ref, b_ref, att_ref, pw_ref, sq_ref):
    phen = phen_ref[...]
    att = jnp.dot(phen, w_ref[...], preferred_element_type=jnp.float32)
    att = att + b_ref[...]
    att_ref[...] = att
    pw = att * phen
    pw_ref[...] = pw
    sq_ref[...] = jnp.sum(pw * pw, axis=1, keepdims=True)


def _topk_kernel(n, rb, pwb_ref, pwt_ref, sqb_ref, sqr_ref, et_ref,
                 vals_ref, idx_ref, lq_ref):
    g = pl.program_id(0)
    pwb = pwb_ref[...]
    dot = jnp.dot(pwb, pwt_ref[...], preferred_element_type=jnp.float32)
    dm = (sqb_ref[...] + sqr_ref[...]) - 2.0 * dot
    dm = jnp.maximum(dm, 0.0)
    logits = dm * et_ref[0, 0]

    row0 = (g * rb).astype(jnp.uint32)
    rows = jax.lax.broadcasted_iota(jnp.uint32, (rb, n), 0) + row0
    cols = jax.lax.broadcasted_iota(jnp.uint32, (rb, n), 1)
    flat = rows * jnp.uint32(n) + cols
    bits = _threefry_bits(flat)
    fb = (bits >> jnp.uint32(9)) | jnp.uint32(0x3F800000)
    u = jax.lax.bitcast_convert_type(fb, jnp.float32) - 1.0
    q = u + jnp.float32(1e-8)
    lq = logits - jnp.log(-jnp.log(q))
    lq_ref[...] = lq

    colidx = jax.lax.broadcasted_iota(jnp.int32, (rb, n), 1)
    vals = []
    idxs = []
    for _ in range(_K):
        v = lq_ref[...]
        m = jnp.min(v, axis=1, keepdims=True)
        ix = jnp.min(jnp.where(v == m, colidx, jnp.int32(n)), axis=1,
                     keepdims=True)
        vals.append(-m)
        idxs.append(ix)
        lq_ref[...] = jnp.where(colidx == ix, jnp.float32(jnp.inf), v)
    vals_ref[...] = jnp.concatenate(vals, axis=1)
    idx_ref[...] = jnp.concatenate(idxs, axis=1)


@jax.jit
def kernel(x, A, phenotypes, W, b, temperature):
    n, d = phenotypes.shape[1], phenotypes.shape[2]
    phen = phenotypes[0]

    att, pw, sq_col = pl.pallas_call(
        _embed_kernel,
        out_shape=[
            jax.ShapeDtypeStruct((n, d), jnp.float32),
            jax.ShapeDtypeStruct((n, d), jnp.float32),
            jax.ShapeDtypeStruct((n, 1), jnp.float32),
        ],
    )(phen, W, b.reshape(1, d))

    pwt = pw.T
    sq_row = sq_col.reshape(1, n)
    scale = jnp.exp(jnp.clip(temperature, -5.0, 5.0)).reshape(1, 1)

    rb = 200
    grid = n // rb
    vals, idx = pl.pallas_call(
        functools.partial(_topk_kernel, n, rb),
        grid=(grid,),
        in_specs=[
            pl.BlockSpec((rb, d), lambda g: (g, 0)),
            pl.BlockSpec((d, n), lambda g: (0, 0)),
            pl.BlockSpec((rb, 1), lambda g: (g, 0)),
            pl.BlockSpec((1, n), lambda g: (0, 0)),
            pl.BlockSpec((1, 1), lambda g: (0, 0)),
        ],
        out_specs=[
            pl.BlockSpec((rb, _K), lambda g: (g, 0)),
            pl.BlockSpec((rb, _K), lambda g: (g, 0)),
        ],
        out_shape=[
            jax.ShapeDtypeStruct((n, _K), jnp.float32),
            jax.ShapeDtypeStruct((n, _K), jnp.int32),
        ],
        scratch_shapes=[pltpu.VMEM((rb, n), jnp.float32)],
    )(pw, pwt, sq_col, sq_row, scale)

    rows = jnp.broadcast_to(jnp.arange(n, dtype=jnp.int32)[:, None], (n, _K))
    edges_hat = jnp.stack([idx.reshape(-1), rows.reshape(-1)], axis=0)
    logprobs = vals.reshape(1, n, _K)
    return (x, edges_hat, phenotypes, logprobs, att.reshape(1, n, d))
